# baseline (speedup 1.0000x reference)
import dataclasses

import jax
import jax.numpy as jnp
from jax import lax
from jax.experimental import pallas as pl
from jax.experimental.pallas import tpu as pltpu
from jax.experimental.pallas import tpu_sc as plsc

VMIN = -5.0
VMAX = 5.0
N = 100
H = (VMAX - VMIN) / (N - 1)
INV_H = (N - 1) / (VMAX - VMIN)
OFF = 1.0 - VMIN * INV_H

NC = 2
NS = 16
NW = NC * NS
L = 16
UNROLL = 4
STEP = L * UNROLL
TBL = 112
PTS_OFF = 112
B_OFF = 224
PARBUF = 240


def _sc_pwl(eps_flat, p, points, b, per_w, last_w):
    mesh = plsc.VectorSubcoreMesh(
        core_axis_name="c", subcore_axis_name="s", num_cores=NC, num_subcores=NS
    )
    total = eps_flat.shape[0]

    NCHUNK = 4

    def body(p_hbm, pts_hbm, b_hbm, eps_hbm, out_hbm,
             par_v, k_v, d_v, c_v, eps_v, out_v,
             psem, sem0, sem1, sem2, sem3):
        csems = [sem0, sem1, sem2, sem3]
        hp = [pltpu.async_copy(p_hbm, par_v.at[pl.ds(0, 101)], psem),
              pltpu.async_copy(pts_hbm, par_v.at[pl.ds(PTS_OFF, N)], psem),
              pltpu.async_copy(b_hbm, par_v.at[pl.ds(B_OFF, 1)], psem)]

        wid = lax.axis_index("s") * NC + lax.axis_index("c")
        base = wid * per_w

        def build_tables():
            for h in hp:
                h.wait()
            bval = par_v[pl.ds(B_OFF, L)][0]

            for i in range(TBL // L):
                k_v[pl.ds(i * L, L)] = jnp.exp(par_v[pl.ds(i * L, L)]) + 0.001

            acc = bval
            for i in range(TBL // L):
                ii = lax.iota(jnp.int32, L) + (i * L)
                g = jnp.where((ii >= 1) & (ii <= N - 1),
                              k_v[pl.ds(i * L, L)] * H, 0.0)
                d_v[pl.ds(i * L, L)] = plsc.cumsum(g) + acc
                acc = acc + jnp.sum(g)

            for i in range(TBL // L):
                ii = lax.iota(jnp.int32, L) + (i * L)
                start = jnp.clip(ii - 1, 0, N - 1)
                dbg = plsc.load_gather(d_v, [start])
                spg = plsc.load_gather(par_v, [start + PTS_OFF])
                c_v[pl.ds(i * L, L)] = dbg - spg * k_v[pl.ds(i * L, L)]

        def do_chunk(c0):
            v = eps_v[pl.ds(c0, L)]
            u = jnp.clip(v * INV_H + OFF, 0.0, float(N))
            idx = u.astype(jnp.int32)
            a = plsc.load_gather(k_v, [idx])
            cc = plsc.load_gather(c_v, [idx])
            out_v[pl.ds(c0, L)] = v * a + cc

        def run_loop(lo, hi):
            main = lo + (hi - lo) // STEP * STEP

            @pl.loop(lo, main, step=STEP)
            def _(c0):
                for j in range(UNROLL):
                    do_chunk(c0 + j * L)

            if hi > main:
                @pl.loop(main, hi, step=L)
                def _(c0):
                    do_chunk(c0)

        def span(base, count):
            edges = sorted({i * count // NCHUNK // STEP * STEP
                            for i in range(NCHUNK)} | {count})
            spans = [(lo, hi) for lo, hi in zip(edges, edges[1:]) if hi > lo]

            ins = [pltpu.async_copy(eps_hbm.at[pl.ds(base + lo, hi - lo)],
                                    eps_v.at[pl.ds(lo, hi - lo)], csems[i])
                   for i, (lo, hi) in enumerate(spans)]
            build_tables()
            outs = []
            for i, (lo, hi) in enumerate(spans):
                ins[i].wait()
                run_loop(lo, hi)
                outs.append(
                    pltpu.async_copy(out_v.at[pl.ds(lo, hi - lo)],
                                     out_hbm.at[pl.ds(base + lo, hi - lo)],
                                     csems[i]))
            for h in outs:
                h.wait()

        if last_w == per_w:
            span(base, per_w)
        else:
            @pl.when(wid < NW - 1)
            def _():
                span(base, per_w)

            @pl.when(wid == NW - 1)
            def _():
                span((NW - 1) * per_w, last_w)

    cp = pltpu.CompilerParams()
    if "needs_layout_passes" in pltpu.CompilerParams.__dataclass_fields__:
        cp = dataclasses.replace(cp, needs_layout_passes=False)
    if "use_tc_tiling_on_sc" in pltpu.CompilerParams.__dataclass_fields__:
        cp = dataclasses.replace(cp, use_tc_tiling_on_sc=False)

    run = pl.kernel(
        body,
        out_type=jax.ShapeDtypeStruct((total,), jnp.float32),
        mesh=mesh,
        compiler_params=cp,
        scratch_types=[
            pltpu.VMEM((PARBUF,), jnp.float32),
            pltpu.VMEM((TBL,), jnp.float32),
            pltpu.VMEM((TBL,), jnp.float32),
            pltpu.VMEM((TBL,), jnp.float32),
            pltpu.VMEM((per_w,), jnp.float32),
            pltpu.VMEM((per_w,), jnp.float32),
            pltpu.SemaphoreType.DMA,
            pltpu.SemaphoreType.DMA,
            pltpu.SemaphoreType.DMA,
            pltpu.SemaphoreType.DMA,
            pltpu.SemaphoreType.DMA,
        ],
    )
    return run(p, points, b, eps_flat)


@jax.jit
def kernel(eps, p, b, points):
    bsz = eps.shape[0]
    pts = points.reshape(-1)

    eps_flat = eps.reshape(-1)
    if bsz % L == 0 and bsz >= NW * L:
        per_w = ((bsz + NW - 1) // NW + L - 1) // L * L
        last_w = bsz - (NW - 1) * per_w
        out = _sc_pwl(eps_flat, p, pts, b, per_w, last_w)
        return out.reshape(bsz, 1)

    chunk = NW * L
    total = ((bsz + chunk - 1) // chunk) * chunk
    per_w = total // NW
    eps_pad = jnp.pad(eps_flat, (0, total - bsz))
    out = _sc_pwl(eps_pad, p, pts, b, per_w, per_w)
    return out[:bsz].reshape(bsz, 1)


# speedup vs baseline: 1.0077x; 1.0077x over previous
import dataclasses

import jax
import jax.numpy as jnp
from jax import lax
from jax.experimental import pallas as pl
from jax.experimental.pallas import tpu as pltpu
from jax.experimental.pallas import tpu_sc as plsc

VMIN = -5.0
VMAX = 5.0
N = 100
H = (VMAX - VMIN) / (N - 1)
INV_H = (N - 1) / (VMAX - VMIN)
OFF = 1.0 - VMIN * INV_H

NC = 2
NS = 16
NW = NC * NS
L = 16
UNROLL = 4
STEP = L * UNROLL
TBL = 112
PTS_OFF = 112
B_OFF = 224
PARBUF = 240


def _sc_pwl(eps_flat, p, points, b, per_w, last_w):
    mesh = plsc.VectorSubcoreMesh(
        core_axis_name="c", subcore_axis_name="s", num_cores=NC, num_subcores=NS
    )
    total = eps_flat.shape[0]

    NCHUNK = 2

    def body(p_hbm, pts_hbm, b_hbm, eps_hbm, out_hbm,
             par_v, k_v, d_v, c_v, eps_v, out_v,
             psem, sem0, sem1, sem2, sem3):
        csems = [sem0, sem1, sem2, sem3]
        hp = [pltpu.async_copy(p_hbm, par_v.at[pl.ds(0, 101)], psem),
              pltpu.async_copy(pts_hbm, par_v.at[pl.ds(PTS_OFF, N)], psem),
              pltpu.async_copy(b_hbm, par_v.at[pl.ds(B_OFF, 1)], psem)]

        wid = lax.axis_index("s") * NC + lax.axis_index("c")
        base = wid * per_w

        def build_tables():
            for h in hp:
                h.wait()
            bval = par_v[pl.ds(B_OFF, L)][0]

            for i in range(TBL // L):
                k_v[pl.ds(i * L, L)] = jnp.exp(par_v[pl.ds(i * L, L)]) + 0.001

            acc = bval
            for i in range(TBL // L):
                ii = lax.iota(jnp.int32, L) + (i * L)
                g = jnp.where((ii >= 1) & (ii <= N - 1),
                              k_v[pl.ds(i * L, L)] * H, 0.0)
                d_v[pl.ds(i * L, L)] = plsc.cumsum(g) + acc
                acc = acc + jnp.sum(g)

            for i in range(TBL // L):
                ii = lax.iota(jnp.int32, L) + (i * L)
                start = jnp.clip(ii - 1, 0, N - 1)
                dbg = plsc.load_gather(d_v, [start])
                spg = plsc.load_gather(par_v, [start + PTS_OFF])
                c_v[pl.ds(i * L, L)] = dbg - spg * k_v[pl.ds(i * L, L)]

        def do_chunk(c0):
            v = eps_v[pl.ds(c0, L)]
            u = jnp.clip(v * INV_H + OFF, 0.0, float(N))
            idx = u.astype(jnp.int32)
            a = plsc.load_gather(k_v, [idx])
            cc = plsc.load_gather(c_v, [idx])
            out_v[pl.ds(c0, L)] = v * a + cc

        def run_loop(lo, hi):
            main = lo + (hi - lo) // STEP * STEP

            @pl.loop(lo, main, step=STEP)
            def _(c0):
                for j in range(UNROLL):
                    do_chunk(c0 + j * L)

            if hi > main:
                @pl.loop(main, hi, step=L)
                def _(c0):
                    do_chunk(c0)

        def span(base, count):
            edges = sorted({i * count // NCHUNK // STEP * STEP
                            for i in range(NCHUNK)} | {count})
            spans = [(lo, hi) for lo, hi in zip(edges, edges[1:]) if hi > lo]

            ins = [pltpu.async_copy(eps_hbm.at[pl.ds(base + lo, hi - lo)],
                                    eps_v.at[pl.ds(lo, hi - lo)], csems[i])
                   for i, (lo, hi) in enumerate(spans)]
            build_tables()
            outs = []
            for i, (lo, hi) in enumerate(spans):
                ins[i].wait()
                run_loop(lo, hi)
                outs.append(
                    pltpu.async_copy(out_v.at[pl.ds(lo, hi - lo)],
                                     out_hbm.at[pl.ds(base + lo, hi - lo)],
                                     csems[i]))
            for h in outs:
                h.wait()

        if last_w == per_w:
            span(base, per_w)
        else:
            @pl.when(wid < NW - 1)
            def _():
                span(base, per_w)

            @pl.when(wid == NW - 1)
            def _():
                span((NW - 1) * per_w, last_w)

    cp = pltpu.CompilerParams()
    if "needs_layout_passes" in pltpu.CompilerParams.__dataclass_fields__:
        cp = dataclasses.replace(cp, needs_layout_passes=False)
    if "use_tc_tiling_on_sc" in pltpu.CompilerParams.__dataclass_fields__:
        cp = dataclasses.replace(cp, use_tc_tiling_on_sc=False)

    run = pl.kernel(
        body,
        out_type=jax.ShapeDtypeStruct((total,), jnp.float32),
        mesh=mesh,
        compiler_params=cp,
        scratch_types=[
            pltpu.VMEM((PARBUF,), jnp.float32),
            pltpu.VMEM((TBL,), jnp.float32),
            pltpu.VMEM((TBL,), jnp.float32),
            pltpu.VMEM((TBL,), jnp.float32),
            pltpu.VMEM((per_w,), jnp.float32),
            pltpu.VMEM((per_w,), jnp.float32),
            pltpu.SemaphoreType.DMA,
            pltpu.SemaphoreType.DMA,
            pltpu.SemaphoreType.DMA,
            pltpu.SemaphoreType.DMA,
            pltpu.SemaphoreType.DMA,
        ],
    )
    return run(p, points, b, eps_flat)


@jax.jit
def kernel(eps, p, b, points):
    bsz = eps.shape[0]
    pts = points.reshape(-1)

    eps_flat = eps.reshape(-1)
    if bsz % L == 0 and bsz >= NW * L:
        per_w = ((bsz + NW - 1) // NW + L - 1) // L * L
        last_w = bsz - (NW - 1) * per_w
        out = _sc_pwl(eps_flat, p, pts, b, per_w, last_w)
        return out.reshape(bsz, 1)

    chunk = NW * L
    total = ((bsz + chunk - 1) // chunk) * chunk
    per_w = total // NW
    eps_pad = jnp.pad(eps_flat, (0, total - bsz))
    out = _sc_pwl(eps_pad, p, pts, b, per_w, per_w)
    return out[:bsz].reshape(bsz, 1)
